# baseline (speedup 1.0000x reference)
import functools

import jax
import jax.numpy as jnp
from jax import lax
from jax.experimental import pallas as pl
from jax.experimental.pallas import tpu as pltpu
from jax.experimental.pallas import tpu_sc as plsc

BATCH = 4096
HIST = 50
EMBED = 64
TOTAL = BATCH * HIST
NC, NS = 2, 16
NW = NC * NS
B_PER_W = TOTAL // NW
CHUNK = 800
N_CHUNKS = B_PER_W // CHUNK
NBUF = 2

_mesh = plsc.VectorSubcoreMesh(core_axis_name="c", subcore_axis_name="s")


@functools.partial(
    pl.kernel,
    out_type=jax.ShapeDtypeStruct((TOTAL, EMBED), jnp.float32),
    mesh=_mesh,
    scratch_types=[
        pltpu.VMEM((N_CHUNKS, CHUNK), jnp.int32),
        [pltpu.VMEM((CHUNK, EMBED), jnp.float32) for _ in range(NBUF)],
        [pltpu.SemaphoreType.DMA for _ in range(NBUF)],
        [pltpu.SemaphoreType.DMA for _ in range(NBUF)],
    ],
    compiler_params=pltpu.CompilerParams(use_tc_tiling_on_sc=False),
)
def _gather_kernel(idx_hbm, table_hbm, out_hbm, idx_v, rows, gsem, ssem):
    wid = lax.axis_index("s") * NC + lax.axis_index("c")
    base = wid * B_PER_W

    pltpu.sync_copy(idx_hbm.at[wid], idx_v)

    def gather(k):
        b = k % NBUF
        return pltpu.async_copy(table_hbm.at[idx_v.at[k]], rows[b], gsem[b])

    gathers = [None] * N_CHUNKS
    stores = [None] * N_CHUNKS
    gathers[0] = gather(0)
    for k in range(N_CHUNKS):
        b = k % NBUF
        gathers[k].wait()
        stores[k] = pltpu.async_copy(
            rows[b], out_hbm.at[pl.ds(base + k * CHUNK, CHUNK)], ssem[b])
        if k + 1 < N_CHUNKS:
            if k >= 1:
                stores[k - 1].wait()
            gathers[k + 1] = gather(k + 1)
    for k in range(N_CHUNKS - 2, N_CHUNKS):
        stores[k].wait()


def kernel(data, table):
    idx = data.reshape(NW, N_CHUNKS, CHUNK).astype(jnp.int32)
    out = _gather_kernel(idx, table)
    return out.reshape(BATCH, HIST, EMBED)


# speedup vs baseline: 1.0083x; 1.0083x over previous
import functools

import jax
import jax.numpy as jnp
from jax import lax
from jax.experimental import pallas as pl
from jax.experimental.pallas import tpu as pltpu
from jax.experimental.pallas import tpu_sc as plsc

BATCH = 4096
HIST = 50
EMBED = 64
TOTAL = BATCH * HIST
NC, NS = 2, 16
NW = NC * NS
B_PER_W = TOTAL // NW
CHUNK = 640
N_CHUNKS = B_PER_W // CHUNK
NBUF = 3

_mesh = plsc.VectorSubcoreMesh(core_axis_name="c", subcore_axis_name="s")


@functools.partial(
    pl.kernel,
    out_type=jax.ShapeDtypeStruct((TOTAL, EMBED), jnp.float32),
    mesh=_mesh,
    scratch_types=[
        pltpu.VMEM((N_CHUNKS, CHUNK), jnp.int32),
        [pltpu.VMEM((CHUNK, EMBED), jnp.float32) for _ in range(NBUF)],
        [pltpu.SemaphoreType.DMA for _ in range(NBUF)],
        [pltpu.SemaphoreType.DMA for _ in range(NBUF)],
    ],
    compiler_params=pltpu.CompilerParams(
        use_tc_tiling_on_sc=False, skip_device_barrier=True),
)
def _gather_kernel(idx_hbm, table_hbm, out_hbm, idx_v, rows, gsem, ssem):
    wid = lax.axis_index("s") * NC + lax.axis_index("c")
    base = wid * B_PER_W

    pltpu.sync_copy(idx_hbm.at[wid], idx_v)

    def gather(k):
        b = k % NBUF
        return pltpu.async_copy(table_hbm.at[idx_v.at[k]], rows[b], gsem[b])

    gathers = [None] * N_CHUNKS
    stores = [None] * N_CHUNKS
    gathers[0] = gather(0)
    gathers[1] = gather(1)
    for k in range(N_CHUNKS):
        b = k % NBUF
        gathers[k].wait()
        stores[k] = pltpu.async_copy(
            rows[b], out_hbm.at[pl.ds(base + k * CHUNK, CHUNK)], ssem[b])
        if k + 2 < N_CHUNKS:
            if k >= 1:
                stores[k - 1].wait()
            gathers[k + 2] = gather(k + 2)
    for k in range(N_CHUNKS - 3, N_CHUNKS):
        stores[k].wait()


def kernel(data, table):
    idx = data.reshape(NW, N_CHUNKS, CHUNK).astype(jnp.int32)
    out = _gather_kernel(idx, table)
    return out.reshape(BATCH, HIST, EMBED)
